# baseline (speedup 1.0000x reference)
import functools

import jax
import jax.numpy as jnp
from jax import lax
from jax.experimental import pallas as pl
from jax.experimental.pallas import tpu as pltpu
from jax.experimental.pallas import tpu_sc as plsc

NC = 2
NS = 16
NW = NC * NS
LANES = 16
NP2 = 10240
HALF = NP2 // NC
NB = 64
BKN = HALF // NB
BPT = NB // NS
NR = 167936


def _zero_f32(ref, n):
  def body(i, carry):
    ref[pl.ds(i * LANES, LANES)] = jnp.zeros((LANES,), jnp.float32)
    return carry
  lax.fori_loop(0, n // LANES, body, 0)


def _make_hist_kernel(e_pad):
  ch = 2048
  per_t = e_pad // NS
  n_chunks = per_t // ch
  slice_pw = NP2 // NS
  mesh = plsc.VectorSubcoreMesh(core_axis_name="c", subcore_axis_name="s")

  @functools.partial(
      pl.kernel,
      out_type=[
          jax.ShapeDtypeStruct((NC, 1, NP2), jnp.float32),
          jax.ShapeDtypeStruct((NC, NS, 1, NB), jnp.int32),
      ],
      mesh=mesh,
      scratch_types=[
          pltpu.VMEM((ch,), jnp.int32),
          pltpu.VMEM((ch // 128, 1, 128), jnp.int32),
          pltpu.VMEM((ch,), jnp.float32),
          pltpu.VMEM((slice_pw,), jnp.float32),
          pltpu.VMEM((NB,), jnp.int32),
          pltpu.VMEM_SHARED((NP2,), jnp.float32),
          pltpu.SemaphoreType.DMA,
      ],
      compiler_params=pltpu.CompilerParams(needs_layout_passes=False),
  )
  def hist_kernel(src_hbm, dst3_hbm, w_hbm, deg_hbm, cnt_hbm,
                  srcv, dstv, wv, zbuf, cnts, acc, sem):
    del sem
    c = lax.axis_index("c")
    s = lax.axis_index("s")
    lo = c * HALF
    _zero_f32(zbuf, slice_pw)
    pltpu.sync_copy(zbuf, acc.at[pl.ds(s * slice_pw, slice_pw)])
    for i in range(NB // LANES):
      cnts[pl.ds(i * LANES, LANES)] = jnp.zeros((LANES,), jnp.int32)
    plsc.subcore_barrier()

    iota = lax.iota(jnp.int32, LANES)
    ones = jnp.ones((LANES,), jnp.int32)

    def chunk(k, carry):
      base = s * per_t + k * ch
      pltpu.sync_copy(src_hbm.at[pl.ds(base, ch)], srcv)
      pltpu.sync_copy(w_hbm.at[pl.ds(base, ch)], wv)
      pltpu.sync_copy(dst3_hbm.at[pl.ds(base // 128, ch // 128)], dstv)

      @pl.when(c == 0)
      def _():
        def sub(j, carry2):
          pltpu.sync_copy(wv.at[pl.ds(j * 128, 128)], acc.at[dstv.at[j, 0]],
                          add=True)
          return carry2
        lax.fori_loop(0, ch // 128, sub, 0)

      def hv(v, carry2):
        sv = srcv[pl.ds(pl.multiple_of(v * LANES, 16), LANES)]
        rel = sv - lo
        maskh = (rel >= 0) & (rel < HALF)
        bk = lax.div(rel, BKN)
        bk = jnp.minimum(jnp.maximum(bk, 0), NB - 1)
        for l in range(LANES):
          m = maskh & (iota == l)
          plsc.addupdate_scatter(cnts, [bk], ones, mask=m)
        return carry2
      lax.fori_loop(0, ch // LANES, hv, 0)
      return carry
    lax.fori_loop(0, n_chunks, chunk, 0)

    plsc.subcore_barrier()
    pltpu.sync_copy(acc.at[pl.ds(s * slice_pw, slice_pw)],
                    deg_hbm.at[c, 0, pl.ds(s * slice_pw, slice_pw)])
    pltpu.sync_copy(cnts, cnt_hbm.at[c, s, 0])

  return hist_kernel


def _mm_body(x_ref, w_ref, d0_ref, d1_ref, cnt_ref,
             xw2_ref, dis_ref, base_ref, bst_ref, ebd_ref):
  deg = d0_ref[...] + d1_ref[...] + 1.0
  dis = lax.rsqrt(deg)
  xw = jnp.dot(x_ref[...], w_ref[...],
               preferred_element_type=jnp.float32,
               precision=lax.Precision.HIGHEST)
  xw2_ref[...] = xw * dis
  dis_ref[...] = dis

  cnt = cnt_ref[...]
  tot = jnp.sum(cnt, axis=1)
  totp = jnp.bitwise_and(tot + 7, jnp.int32(-8))
  inc = totp
  sh = 1
  while sh < NB:
    z = jnp.zeros((NC, sh), jnp.int32)
    inc = inc + jnp.concatenate([z, inc[:, :-sh]], axis=1)
    sh *= 2
  bstart = inc - totp
  incT = cnt
  sh = 1
  while sh < NS:
    zT = jnp.zeros((NC, sh, NB), jnp.int32)
    incT = incT + jnp.concatenate([zT, incT[:, :-sh, :]], axis=1)
    sh *= 2
  exclT = incT - cnt
  base_ref[...] = (bstart[:, None, :] + exclT).reshape(NC, NS, 1, NB)
  bst_ref[...] = jnp.broadcast_to(bstart[:, :, None, None],
                                  (NC, NB, 1, 128)).astype(jnp.int32)
  ebd_ref[...] = jnp.broadcast_to((bstart + tot)[:, :, None, None],
                                  (NC, NB, 1, 128)).astype(jnp.int32)


def _make_bc_kernel(e_pad, d):
  ch = 1024
  per_t = e_pad // NS
  n_chunks = per_t // ch
  rows_pw = NP2 // NS
  mesh = plsc.VectorSubcoreMesh(core_axis_name="c", subcore_axis_name="s")

  @functools.partial(
      pl.kernel,
      out_type=jax.ShapeDtypeStruct((NC, NP2, d), jnp.float32),
      mesh=mesh,
      scratch_types=[
          pltpu.VMEM((2, ch), jnp.int32),
          pltpu.VMEM((2, ch), jnp.int32),
          pltpu.VMEM((2, ch), jnp.float32),
          pltpu.VMEM((ch,), jnp.int32),
          pltpu.VMEM((ch // 128, 1, 128), jnp.int32),
          pltpu.VMEM((NB,), jnp.int32),
          pltpu.VMEM((128,), jnp.int32),
          pltpu.VMEM((128,), jnp.int32),
          pltpu.VMEM((BKN, d), jnp.float32),
          pltpu.VMEM((1024,), jnp.int32),
          pltpu.VMEM((1024,), jnp.float32),
          pltpu.VMEM((LANES, d), jnp.float32),
          pltpu.VMEM((LANES, d), jnp.float32),
          pltpu.VMEM((1, 1, LANES), jnp.int32),
          pltpu.VMEM((1, 1, LANES), jnp.int32),
          pltpu.VMEM_SHARED((NP2, d), jnp.float32),
          pltpu.VMEM_SHARED((NR,), jnp.int32),
          pltpu.VMEM_SHARED((NR,), jnp.float32),
          pltpu.SemaphoreType.DMA,
          pltpu.SemaphoreType.DMA,
          pltpu.SemaphoreType.DMA,
          pltpu.SemaphoreType.DMA,
          pltpu.SemaphoreType.DMA,
      ],
      compiler_params=pltpu.CompilerParams(needs_layout_passes=False),
  )
  def bc_kernel(src_hbm, dst_hbm, w_hbm, xw2_hbm, base_hbm, bst_hbm, ebd_hbm,
                out_hbm, srcv, dstv, wv, packedv, slotv3, cnts, bstb, ebdb,
                block, recp, recw, stag0, stag1, didx0, didx1,
                acc, recps, recws, isem, rsem, ssem0, ssem1, msem):
    c = lax.axis_index("c")
    s = lax.axis_index("s")
    lo = c * HALF

    def zrow(i, carry):
      e = i // (d // LANES)
      j = i % (d // LANES)
      stag0[e, pl.ds(pl.multiple_of(j * LANES, 16), LANES)] = (
          jnp.zeros((LANES,), jnp.float32))
      return carry
    lax.fori_loop(0, LANES * d // LANES, zrow, 0)
    for off in range(0, rows_pw, LANES):
      pltpu.sync_copy(
          stag0, acc.at[pl.ds(pl.multiple_of(s * rows_pw + off, 8), LANES)])

    pltpu.sync_copy(base_hbm.at[c, s, 0], cnts)
    plsc.subcore_barrier()

    iota = lax.iota(jnp.int32, LANES)
    ones = jnp.ones((LANES,), jnp.int32)

    def scan_copies(k, parity):
      base = s * per_t + k * ch
      return [
          pltpu.make_async_copy(src_hbm.at[pl.ds(base, ch)],
                                srcv.at[parity], isem),
          pltpu.make_async_copy(dst_hbm.at[pl.ds(base, ch)],
                                dstv.at[parity], isem),
          pltpu.make_async_copy(w_hbm.at[pl.ds(base, ch)],
                                wv.at[parity], isem),
      ]

    for cp in scan_copies(0, 0):
      cp.start()

    def chunk(k, carry):
      parity = lax.rem(k, 2)
      for cp in scan_copies(k, parity):
        cp.wait()

      @pl.when(k + 1 < n_chunks)
      def _():
        for cp in scan_copies(k + 1, 1 - parity):
          cp.start()

      def hv(v, carry2):
        off = pl.multiple_of(v * LANES, 16)
        sv = srcv[parity, pl.ds(off, LANES)]
        dv = dstv[parity, pl.ds(off, LANES)]
        rel = sv - lo
        maskh = (rel >= 0) & (rel < HALF)
        bk = lax.div(rel, BKN)
        bk = jnp.minimum(jnp.maximum(bk, 0), NB - 1)
        packed16 = rel * 16384 + dv
        slotv = jnp.full((LANES,), NR - 1, jnp.int32)
        for l in range(LANES):
          m = maskh & (iota == l)
          cv = plsc.load_gather(cnts, [bk], mask=m)
          slotv = jnp.where(m, cv, slotv)
          plsc.addupdate_scatter(cnts, [bk], ones, mask=m)
        packedv[pl.ds(off, LANES)] = packed16
        slotv3[lax.div(v, 8), 0,
               pl.ds(pl.multiple_of(lax.rem(v, 8) * LANES, 16), LANES)] = slotv
        return carry2
      lax.fori_loop(0, ch // LANES, hv, 0)

      descs = []
      for row in range(ch // 128):
        d1 = pltpu.make_async_copy(packedv.at[pl.ds(row * 128, 128)],
                                   recps.at[slotv3.at[row, 0]], msem)
        d2 = pltpu.make_async_copy(wv.at[parity, pl.ds(row * 128, 128)],
                                   recws.at[slotv3.at[row, 0]], msem)
        d1.start()
        d2.start()
        descs += [d1, d2]
      for dd in descs:
        dd.wait()
      return carry
    lax.fori_loop(0, n_chunks, chunk, 0)
    plsc.subcore_barrier()

    for bkt in range(BPT):
      b0 = s * BPT + bkt
      pltpu.sync_copy(bst_hbm.at[c, b0, 0], bstb)
      pltpu.sync_copy(ebd_hbm.at[c, b0, 0], ebdb)
      st = pl.multiple_of(bstb[pl.ds(0, LANES)][0], 8)
      en = ebdb[pl.ds(0, LANES)][0]
      row_lo = pl.multiple_of(lo + b0 * BKN, 8)
      pltpu.sync_copy(xw2_hbm.at[pl.ds(row_lo, BKN)], block)
      rel_lo = b0 * BKN

      @pl.when(st < en)
      def _():
        pltpu.make_async_copy(recps.at[pl.ds(st, 512)],
                              recp.at[pl.ds(0, 512)], rsem).start()
        pltpu.make_async_copy(recws.at[pl.ds(st, 512)],
                              recw.at[pl.ds(0, 512)], rsem).start()

      def build(par, rbase, gi, stagX, didxX):
        gio = pl.multiple_of(par * 512 + gi, 16)
        p16 = recp[pl.ds(gio, LANES)]
        w16 = recw[pl.ds(gio, LANES)]
        valid = (rbase + gi + iota) < en
        d16 = jnp.where(valid, p16 & 16383, NP2 - 1)
        didxX[0, 0, :] = d16
        for l in range(LANES):
          pe = p16[l]
          loc = jnp.minimum(
              jnp.maximum(lax.shift_right_arithmetic(pe, 14) - rel_lo, 0),
              BKN - 1)
          we = w16[l]
          for j in range(d // LANES):
            stagX[l, pl.ds(j * LANES, LANES)] = (
                block[loc, pl.ds(j * LANES, LANES)] * we)

      def body(carry):
        r, par = carry
        ra = pl.multiple_of(r, 8)
        po = pl.multiple_of(par * 512, 128)
        pltpu.make_async_copy(recps.at[pl.ds(ra, 512)],
                              recp.at[pl.ds(po, 512)], rsem).wait()
        pltpu.make_async_copy(recws.at[pl.ds(ra, 512)],
                              recw.at[pl.ds(po, 512)], rsem).wait()
        r2 = r + 512

        @pl.when(r2 < en)
        def _():
          r2a = pl.multiple_of(r2, 8)
          po2 = pl.multiple_of((1 - par) * 512, 128)
          pltpu.make_async_copy(recps.at[pl.ds(r2a, 512)],
                                recp.at[pl.ds(po2, 512)], rsem).start()
          pltpu.make_async_copy(recws.at[pl.ds(r2a, 512)],
                                recw.at[pl.ds(po2, 512)], rsem).start()

        def pair(q, carry2):
          gi = q * 2 * LANES

          @pl.when(q > 0)
          def _():
            pltpu.make_async_copy(stag0, acc.at[didx0.at[0, 0]],
                                  ssem0).wait()
          build(par, r, gi, stag0, didx0)
          pltpu.make_async_copy(stag0, acc.at[didx0.at[0, 0]],
                                ssem0).start(add=True)

          @pl.when(q > 0)
          def _():
            pltpu.make_async_copy(stag1, acc.at[didx1.at[0, 0]],
                                  ssem1).wait()
          build(par, r, gi + LANES, stag1, didx1)
          pltpu.make_async_copy(stag1, acc.at[didx1.at[0, 0]],
                                ssem1).start(add=True)
          return carry2
        lax.fori_loop(0, 512 // (2 * LANES), pair, 0)
        pltpu.make_async_copy(stag0, acc.at[didx0.at[0, 0]], ssem0).wait()
        pltpu.make_async_copy(stag1, acc.at[didx1.at[0, 0]], ssem1).wait()
        return (r2, 1 - par)

      lax.while_loop(lambda cr: cr[0] < en, body, (st, 0))

    plsc.subcore_barrier()
    pltpu.sync_copy(acc.at[pl.ds(s * rows_pw, rows_pw)],
                    out_hbm.at[c, pl.ds(s * rows_pw, rows_pw)])

  return bc_kernel


def _bn_body(a0_ref, a1_ref, xw2_ref, dis_ref, b_ref, g_ref, be_ref, out_ref):
  n = a0_ref.shape[0]
  t = (a0_ref[...] + a1_ref[...] + xw2_ref[...]) * dis_ref[...] + b_ref[...]
  mean = jnp.sum(t, axis=0, keepdims=True) * (1.0 / n)
  tc = t - mean
  var = jnp.sum(tc * tc, axis=0, keepdims=True) * (1.0 / n)
  h = tc * lax.rsqrt(var + 1e-5) * g_ref[...] + be_ref[...]
  out_ref[...] = jnp.maximum(h, 0.0)


def kernel(x, edge_index, edge_weight, W, b, gamma, beta):
  n, d_in = x.shape
  d_out = W.shape[1]
  e = edge_weight.shape[0]

  unit = NS * 2048
  e_pad = -(-e // unit) * unit
  pad = e_pad - e
  src_p = jnp.concatenate([edge_index[0], jnp.full((pad,), -1, jnp.int32)])
  dst_p = jnp.concatenate([edge_index[1], jnp.zeros((pad,), jnp.int32)])
  w_p = jnp.concatenate([edge_weight, jnp.zeros((pad,), jnp.float32)])
  dst3 = dst_p.reshape(e_pad // 128, 1, 128)
  xp = jnp.concatenate([x, jnp.zeros((NP2 - n, d_in), jnp.float32)])

  degp, counts = _make_hist_kernel(e_pad)(src_p, dst3, w_p)

  dp0 = degp[0, 0].reshape(NP2, 1)
  dp1 = degp[1, 0].reshape(NP2, 1)
  xw2, dis, base4, bst4, ebd4 = pl.pallas_call(
      _mm_body,
      out_shape=[
          jax.ShapeDtypeStruct((NP2, d_out), jnp.float32),
          jax.ShapeDtypeStruct((NP2, 1), jnp.float32),
          jax.ShapeDtypeStruct((NC, NS, 1, NB), jnp.int32),
          jax.ShapeDtypeStruct((NC, NB, 1, 128), jnp.int32),
          jax.ShapeDtypeStruct((NC, NB, 1, 128), jnp.int32),
      ],
  )(xp, W, dp0, dp1, counts.reshape(NC, NS, NB))

  accp = _make_bc_kernel(e_pad, d_out)(src_p, dst_p, w_p, xw2, base4,
                                       bst4, ebd4)

  out = pl.pallas_call(
      _bn_body,
      out_shape=jax.ShapeDtypeStruct((n, d_out), jnp.float32),
  )(accp[0, :n], accp[1, :n], xw2[:n], dis[:n],
    b.reshape(1, d_out), gamma.reshape(1, d_out), beta.reshape(1, d_out))
  return out


# speedup vs baseline: 1.4134x; 1.4134x over previous
import functools

import jax
import jax.numpy as jnp
from jax import lax
from jax.experimental import pallas as pl
from jax.experimental.pallas import tpu as pltpu
from jax.experimental.pallas import tpu_sc as plsc

NC = 2
NS = 16
NW = NC * NS
LANES = 16


def _zero_f32(ref, n):
  def body(i, carry):
    ref[pl.ds(i * LANES, LANES)] = jnp.zeros((LANES,), jnp.float32)
    return carry
  lax.fori_loop(0, n // LANES, body, 0)


def _make_deg_kernel(e_pad, np_):
  ch = 2048
  per_w = e_pad // NW
  n_chunks = per_w // ch
  slice_pw = np_ // NS
  mesh = plsc.VectorSubcoreMesh(core_axis_name="c", subcore_axis_name="s")

  @functools.partial(
      pl.kernel,
      out_type=jax.ShapeDtypeStruct((NC, 1, np_), jnp.float32),
      mesh=mesh,
      scratch_types=[
          pltpu.VMEM((ch // 128, 1, 128), jnp.int32),
          pltpu.VMEM((ch,), jnp.float32),
          pltpu.VMEM((slice_pw,), jnp.float32),
          pltpu.VMEM_SHARED((np_,), jnp.float32),
          pltpu.SemaphoreType.DMA,
      ],
  )
  def deg_kernel(dst3_hbm, w_hbm, out_hbm, dstv, wv, zbuf, acc, sem):
    del sem
    c = lax.axis_index("c")
    s = lax.axis_index("s")
    wid = s * NC + c
    _zero_f32(zbuf, slice_pw)
    pltpu.sync_copy(zbuf, acc.at[pl.ds(s * slice_pw, slice_pw)])
    plsc.subcore_barrier()

    def chunk(k, carry):
      base = wid * per_w + k * ch
      pltpu.sync_copy(w_hbm.at[pl.ds(base, ch)], wv)
      pltpu.sync_copy(dst3_hbm.at[pl.ds(base // 128, ch // 128)], dstv)

      def sub(j, carry2):
        pltpu.sync_copy(wv.at[pl.ds(j * 128, 128)], acc.at[dstv.at[j, 0]],
                        add=True)
        return carry2
      lax.fori_loop(0, ch // 128, sub, 0)
      return carry
    lax.fori_loop(0, n_chunks, chunk, 0)

    plsc.subcore_barrier()
    pltpu.sync_copy(acc.at[pl.ds(s * slice_pw, slice_pw)],
                    out_hbm.at[c, 0, pl.ds(s * slice_pw, slice_pw)])

  return deg_kernel


def _make_msg_kernel(e_pad, np_, d):
  blk = 1024
  p = 64
  npc = blk // p
  nbuf = 4
  per_w = e_pad // NW
  n_blocks = per_w // blk
  rows_pw = np_ // NS
  mesh = plsc.VectorSubcoreMesh(core_axis_name="c", subcore_axis_name="s")

  @functools.partial(
      pl.kernel,
      out_type=jax.ShapeDtypeStruct((NC, np_, d), jnp.float32),
      mesh=mesh,
      scratch_types=[
          pltpu.VMEM((2, blk), jnp.int32),
          pltpu.VMEM((2, npc, 1, p), jnp.int32),
          pltpu.VMEM((2, blk), jnp.float32),
          [pltpu.VMEM((p, d), jnp.float32) for _ in range(nbuf)],
          pltpu.VMEM_SHARED((np_, d), jnp.float32),
          [pltpu.SemaphoreType.DMA for _ in range(nbuf)],
          [pltpu.SemaphoreType.DMA for _ in range(nbuf)],
          pltpu.SemaphoreType.DMA,
      ],
  )
  def msg_kernel(src_hbm, dst4_hbm, w_hbm, xw2_hbm, out_hbm,
                 srcv, dstv, wv, gbufs, acc, gsems, ssems, isem):
    c = lax.axis_index("c")
    s = lax.axis_index("s")
    wid = s * NC + c

    def zrow(i, carry):
      e = i // (d // LANES)
      j = i % (d // LANES)
      gbufs[0][e, pl.ds(j * LANES, LANES)] = jnp.zeros((LANES,), jnp.float32)
      return carry
    lax.fori_loop(0, p * d // LANES, zrow, 0)
    for off in range(0, rows_pw, p):
      pltpu.sync_copy(gbufs[0], acc.at[pl.ds(s * rows_pw + off, p)])
    plsc.subcore_barrier()

    def idx_copies(k, parity):
      ebase = wid * per_w + k * blk
      return [
          pltpu.make_async_copy(src_hbm.at[pl.ds(ebase, blk)],
                                srcv.at[parity], isem),
          pltpu.make_async_copy(w_hbm.at[pl.ds(ebase, blk)],
                                wv.at[parity], isem),
          pltpu.make_async_copy(dst4_hbm.at[pl.ds(ebase // p, npc)],
                                dstv.at[parity], isem),
      ]

    for cp in idx_copies(0, 0):
      cp.start()

    def block(k, carry):
      parity = lax.rem(k, 2)
      for cp in idx_copies(k, parity):
        cp.wait()

      @pl.when(k + 1 < n_blocks)
      def _():
        for cp in idx_copies(k + 1, 1 - parity):
          cp.start()

      def gath(i, b):
        return pltpu.make_async_copy(
            xw2_hbm.at[srcv.at[parity, pl.ds(i * p, p)]], gbufs[b], gsems[b])

      def scat(i, b):
        return pltpu.make_async_copy(gbufs[b], acc.at[dstv.at[parity, i, 0]],
                                     ssems[b])

      gath(0, 0).start()
      gath(1, 1).start()
      sdescs = [None] * npc
      for i in range(npc):
        b = i % nbuf
        gath(i, b).wait()

        def scale(g, carry2):
          wvec = wv[parity, pl.ds(i * p + g * LANES, LANES)]
          e0 = g * LANES
          for l in range(LANES):
            we = wvec[l]
            for j in range(d // LANES):
              gbufs[b][e0 + l, pl.ds(j * LANES, LANES)] = (
                  gbufs[b][e0 + l, pl.ds(j * LANES, LANES)] * we)
          return carry2
        lax.fori_loop(0, p // LANES, scale, 0)

        sd = scat(i, b)
        sd.start(add=True)
        sdescs[i] = sd
        if i + 2 < npc:
          b2 = (i + 2) % nbuf
          if i - 2 >= 0:
            sdescs[i - 2].wait()
          gath(i + 2, b2).start()
      for i in range(npc - 4, npc):
        sdescs[i].wait()
      return carry
    lax.fori_loop(0, n_blocks, block, 0)

    plsc.subcore_barrier()
    pltpu.sync_copy(acc.at[pl.ds(s * rows_pw, rows_pw)],
                    out_hbm.at[c, pl.ds(s * rows_pw, rows_pw)])

  return msg_kernel


def _mm_body(x_ref, w_ref, d0_ref, d1_ref, xw2_ref, dis_ref):
  deg = d0_ref[...] + d1_ref[...] + 1.0
  dis = lax.rsqrt(deg)
  xw = jnp.dot(x_ref[...], w_ref[...],
               preferred_element_type=jnp.float32,
               precision=lax.Precision.HIGHEST)
  xw2_ref[...] = xw * dis
  dis_ref[...] = dis


def _bn_body(a0_ref, a1_ref, xw2_ref, dis_ref, b_ref, g_ref, be_ref, out_ref):
  n = a0_ref.shape[0]
  t = (a0_ref[...] + a1_ref[...] + xw2_ref[...]) * dis_ref[...] + b_ref[...]
  mean = jnp.sum(t, axis=0, keepdims=True) * (1.0 / n)
  tc = t - mean
  var = jnp.sum(tc * tc, axis=0, keepdims=True) * (1.0 / n)
  h = tc * lax.rsqrt(var + 1e-5) * g_ref[...] + be_ref[...]
  out_ref[...] = jnp.maximum(h, 0.0)


def kernel(x, edge_index, edge_weight, W, b, gamma, beta):
  n, d_in = x.shape
  d_out = W.shape[1]
  e = edge_weight.shape[0]

  unit = NW * 2048
  e_pad = -(-e // unit) * unit
  np_ = -(-n // (NS * 640)) * (NS * 640)

  pad = e_pad - e
  src_p = jnp.concatenate([edge_index[0], jnp.zeros((pad,), jnp.int32)])
  dst_p = jnp.concatenate([edge_index[1], jnp.zeros((pad,), jnp.int32)])
  w_p = jnp.concatenate([edge_weight, jnp.zeros((pad,), jnp.float32)])
  dst3 = dst_p.reshape(e_pad // 128, 1, 128)
  dst4 = dst_p.reshape(e_pad // 64, 1, 64)

  degp = _make_deg_kernel(e_pad, np_)(dst3, w_p)

  dp0 = degp[0, 0, :n].reshape(n, 1)
  dp1 = degp[1, 0, :n].reshape(n, 1)
  xw2, dis = pl.pallas_call(
      _mm_body,
      out_shape=[
          jax.ShapeDtypeStruct((n, d_out), jnp.float32),
          jax.ShapeDtypeStruct((n, 1), jnp.float32),
      ],
  )(x, W, dp0, dp1)

  accp = _make_msg_kernel(e_pad, np_, d_out)(src_p, dst4, w_p, xw2)

  out = pl.pallas_call(
      _bn_body,
      out_shape=jax.ShapeDtypeStruct((n, d_out), jnp.float32),
  )(accp[0, :n], accp[1, :n], xw2, dis,
    b.reshape(1, d_out), gamma.reshape(1, d_out), beta.reshape(1, d_out))
  return out
